# baseline (speedup 1.0000x reference)
import functools

import jax
import jax.numpy as jnp
from jax import lax
from jax.experimental import pallas as pl
from jax.experimental.pallas import tpu as pltpu
from jax.experimental.pallas import tpu_sc as plsc

_N = 10000
_NP = 10112
_E = 320000
_NC = 2
_NS = 16
_NW = _NC * _NS
_CH = 128


_EP = 327680
_NIT = _EP // _CH // _NW


def _make_agg(d):
    npt = _NP // _NS
    zr = 128
    mesh = plsc.VectorSubcoreMesh(core_axis_name="c", subcore_axis_name="s")

    @functools.partial(
        pl.kernel,
        mesh=mesh,
        out_type=jax.ShapeDtypeStruct((_NC * _NP, d), jnp.float32),
        scratch_types=[
            pltpu.VMEM((2, _CH), jnp.int32),
            pltpu.VMEM((2, _CH), jnp.int32),
            pltpu.VMEM((_CH, d), jnp.float32),
            pltpu.VMEM((_CH, d), jnp.float32),
            pltpu.VMEM((zr, d), jnp.float32),
            pltpu.VMEM_SHARED((_NP, d), jnp.float32),
            pltpu.SemaphoreType.DMA,
            pltpu.SemaphoreType.DMA,
            pltpu.SemaphoreType.DMA,
            pltpu.SemaphoreType.DMA,
        ],
    )
    def agg(y_hbm, ei_hbm, out_hbm, ib0, ib1, rw0, rw1, zbuf, acc,
            g0, g1, s0, s1):
        cid = lax.axis_index("c")
        sid = lax.axis_index("s")
        tid = sid * _NC + cid
        ibuf = (ib0, ib1)
        rows = (rw0, rw1)
        gsem = (g0, g1)
        ssem = (s0, s1)

        zv = jnp.zeros((16,), jnp.float32)

        def zrow(r, carry):
            for l in range(d // 16):
                zbuf[r, pl.ds(l * 16, 16)] = zv
            return carry

        lax.fori_loop(0, zr, zrow, 0)
        for k in range(npt // zr):
            pltpu.sync_copy(zbuf, acc.at[pl.ds(sid * npt + k * zr, zr)])
        rem = npt % zr
        if rem:
            pltpu.sync_copy(zbuf.at[pl.ds(0, rem)],
                            acc.at[pl.ds(sid * npt + (npt // zr) * zr, rem)])
        plsc.subcore_barrier()

        def idx_load(k, b):
            base = (k * _NW + tid) * _CH
            pltpu.sync_copy(ei_hbm.at[:, pl.ds(base, _CH)], ibuf[b])

        def gather(b):
            return pltpu.async_copy(y_hbm.at[ibuf[b].at[0]], rows[b], gsem[b])

        def gather_wait(b):
            pltpu.make_async_copy(y_hbm.at[ibuf[b].at[0]], rows[b],
                                  gsem[b]).wait()

        def scatter(b):
            pltpu.async_copy(rows[b], acc.at[ibuf[b].at[1]], ssem[b],
                             add=True)

        def scatter_wait(b):
            pltpu.make_async_copy(rows[b], acc.at[ibuf[b].at[1]],
                                  ssem[b]).wait()

        idx_load(0, 0)
        gather(0)

        def body(j, carry):
            @pl.when(j > 0)
            def _():
                scatter_wait(1)
            idx_load(2 * j + 1, 1)
            gather(1)
            gather_wait(0)
            scatter(0)
            @pl.when(j < (_NIT // 2 - 1))
            def _():
                scatter_wait(0)
                idx_load(2 * j + 2, 0)
                gather(0)
            gather_wait(1)
            scatter(1)
            return carry

        lax.fori_loop(0, _NIT // 2, body, 0)
        scatter_wait(0)
        scatter_wait(1)
        plsc.subcore_barrier()
        pltpu.sync_copy(acc.at[pl.ds(sid * npt, npt)],
                        out_hbm.at[pl.ds(cid * _NP + sid * npt, npt)])

    return agg


def _mm_body(x_ref, w_ref, o_ref):
    o_ref[...] = jnp.dot(x_ref[...], w_ref[...],
                         preferred_element_type=jnp.float32)


def _matmul(x, w):
    return pl.pallas_call(
        _mm_body,
        out_shape=jax.ShapeDtypeStruct((x.shape[0], w.shape[1]), jnp.float32),
    )(x, w)


def _psum(p_ref):
    return p_ref[pl.ds(0, _N), :] + p_ref[pl.ds(_NP, _N), :]


def _combine_mm_body(y_ref, p_ref, b_ref, s_ref, w_ref, o_ref):
    h = s_ref[...] * y_ref[...] + _psum(p_ref) + b_ref[...]
    h = jnp.maximum(h, 0.0)
    o_ref[...] = jnp.dot(h, w_ref[...], preferred_element_type=jnp.float32)


def _combine_mm(y, p, b, s, w):
    return pl.pallas_call(
        _combine_mm_body,
        out_shape=jax.ShapeDtypeStruct((y.shape[0], w.shape[1]), jnp.float32),
    )(y, p, b, s, w)


def _combine_body(y_ref, p_ref, b_ref, s_ref, o_ref):
    h = s_ref[...] * y_ref[...] + _psum(p_ref) + b_ref[...]
    o_ref[...] = jnp.maximum(h, 0.0)


def _combine(y, p, b, s):
    return pl.pallas_call(
        _combine_body,
        out_shape=jax.ShapeDtypeStruct(y.shape, jnp.float32),
    )(y, p, b, s)


def _final_body(h_ref, p_ref, b_ref, s_ref, w_ref, lo_ref, pr_ref):
    z = s_ref[...] * h_ref[...] + _psum(p_ref)
    logits = jnp.dot(z, w_ref[...], preferred_element_type=jnp.float32)
    logits = jnp.maximum(logits + b_ref[...], 0.0)
    lo_ref[...] = logits
    m = jnp.max(logits, axis=-1, keepdims=True)
    ex = jnp.exp(logits - m)
    pr_ref[...] = ex / jnp.sum(ex, axis=-1, keepdims=True)


def _final(h, p, b, s, w):
    n = h.shape[0]
    d = w.shape[1]
    return pl.pallas_call(
        _final_body,
        out_shape=(jax.ShapeDtypeStruct((n, d), jnp.float32),
                   jax.ShapeDtypeStruct((n, d), jnp.float32)),
    )(h, p, b, s, w)


@jax.jit
def kernel(x, edge_index, W1, b1, eps1, W2, b2, eps2, W3, b3, eps3):
    pad = jnp.concatenate(
        [jnp.zeros((1, _EP - _E), jnp.int32),
         jnp.full((1, _EP - _E), _NP - 1, jnp.int32)], axis=0)
    ei = jnp.concatenate([edge_index, pad], axis=1)
    s1 = jnp.reshape(1.0 + eps1, (1, 1))
    s2 = jnp.reshape(1.0 + eps2, (1, 1))
    s3 = jnp.reshape(1.0 + eps3, (1, 1))

    agg128 = _make_agg(128)

    y1 = _matmul(x, W1)
    p1 = agg128(y1, ei)
    y2 = _combine_mm(y1, p1, b1.reshape(1, -1), s1, W2)
    p2 = agg128(y2, ei)
    h2 = _combine(y2, p2, b2.reshape(1, -1), s2)
    p3 = agg128(h2, ei)
    logits, probs = _final(h2, p3, b3.reshape(1, -1), s3, W3)
    return (logits, probs)


# speedup vs baseline: 1.0184x; 1.0184x over previous
import functools

import jax
import jax.numpy as jnp
from jax import lax
from jax.experimental import pallas as pl
from jax.experimental.pallas import tpu as pltpu
from jax.experimental.pallas import tpu_sc as plsc

_N = 10000
_NP = 10112
_E = 320000
_NC = 2
_NS = 16
_NW = _NC * _NS
_CH = 128


_EP = 327680
_NIT = _EP // _CH // _NW


def _make_agg(d):
    npt = _NP // _NS
    zr = 128
    mesh = plsc.VectorSubcoreMesh(core_axis_name="c", subcore_axis_name="s")

    ph_n = _NIT // 2
    scratch_types = [
            pltpu.VMEM((2, ph_n, _CH), jnp.int32),
            pltpu.VMEM((_CH, d), jnp.float32),
            pltpu.VMEM((_CH, d), jnp.float32),
            pltpu.VMEM_SHARED((_NP, d), jnp.float32),
            pltpu.SemaphoreType.DMA,
            pltpu.SemaphoreType.DMA,
    ]

    @functools.partial(
        pl.kernel,
        mesh=mesh,
        out_type=jax.ShapeDtypeStruct((_NC * _NP, d), jnp.float32),
        scratch_types=scratch_types,
    )
    def agg(y_hbm, ei_hbm, out_hbm, islab, rw0, rw1, acc, g0, g1):
        cid = lax.axis_index("c")
        sid = lax.axis_index("s")
        tid = sid * _NC + cid
        rows = (rw0, rw1)
        gsem = (g0, g1)

        zv = jnp.zeros((16,), jnp.float32)

        def zrow(r, carry):
            for l in range(d // 16):
                rw0[r, pl.ds(l * 16, 16)] = zv
            return carry

        lax.fori_loop(0, zr, zrow, 0)
        for k in range(npt // zr):
            pltpu.sync_copy(rw0, acc.at[pl.ds(sid * npt + k * zr, zr)])
        rem = npt % zr
        if rem:
            pltpu.sync_copy(rw0.at[pl.ds(0, rem)],
                            acc.at[pl.ds(sid * npt + (npt // zr) * zr, rem)])
        plsc.subcore_barrier()

        def gather(b, c):
            pltpu.async_copy(y_hbm.at[islab.at[0, c]], rows[b], gsem[b])

        def gather_wait(b, c):
            pltpu.make_async_copy(y_hbm.at[islab.at[0, c]], rows[b],
                                  gsem[b]).wait()

        def scatter(b, c):
            pltpu.sync_copy(rows[b], acc.at[islab.at[1, c]], add=True)

        for ph in range(2):
            pltpu.sync_copy(
                ei_hbm.at[:, pl.ds(tid * _NIT + ph * ph_n, ph_n)], islab)
            gather(0, 0)

            def body(j, carry):
                k = 2 * j
                gather(1, k + 1)
                gather_wait(0, k)
                scatter(0, k)

                @pl.when(j < (ph_n // 2 - 1))
                def _():
                    gather(0, k + 2)
                gather_wait(1, k + 1)
                scatter(1, k + 1)
                return carry

            lax.fori_loop(0, ph_n // 2, body, 0)

        plsc.subcore_barrier()
        pltpu.sync_copy(acc.at[pl.ds(sid * npt, npt)],
                        out_hbm.at[pl.ds(cid * _NP + sid * npt, npt)])

    return agg


def _mm_body(x_ref, w_ref, o_ref):
    o_ref[...] = jnp.dot(x_ref[...], w_ref[...],
                         preferred_element_type=jnp.float32)


def _matmul(x, w):
    return pl.pallas_call(
        _mm_body,
        out_shape=jax.ShapeDtypeStruct((x.shape[0], w.shape[1]), jnp.float32),
    )(x, w)


def _psum(p_ref):
    return p_ref[pl.ds(0, _N), :] + p_ref[pl.ds(_NP, _N), :]


def _combine_mm_body(y_ref, p_ref, b_ref, s_ref, w_ref, o_ref):
    h = s_ref[...] * y_ref[...] + _psum(p_ref) + b_ref[...]
    h = jnp.maximum(h, 0.0)
    o_ref[...] = jnp.dot(h, w_ref[...], preferred_element_type=jnp.float32)


def _combine_mm(y, p, b, s, w):
    return pl.pallas_call(
        _combine_mm_body,
        out_shape=jax.ShapeDtypeStruct((y.shape[0], w.shape[1]), jnp.float32),
    )(y, p, b, s, w)


def _combine_body(y_ref, p_ref, b_ref, s_ref, o_ref):
    h = s_ref[...] * y_ref[...] + _psum(p_ref) + b_ref[...]
    o_ref[...] = jnp.maximum(h, 0.0)


def _combine(y, p, b, s):
    return pl.pallas_call(
        _combine_body,
        out_shape=jax.ShapeDtypeStruct(y.shape, jnp.float32),
    )(y, p, b, s)


def _final_body(h_ref, p_ref, b_ref, s_ref, w_ref, lo_ref, pr_ref):
    z = s_ref[...] * h_ref[...] + _psum(p_ref)
    logits = jnp.dot(z, w_ref[...], preferred_element_type=jnp.float32)
    logits = jnp.maximum(logits + b_ref[...], 0.0)
    lo_ref[...] = logits
    m = jnp.max(logits, axis=-1, keepdims=True)
    ex = jnp.exp(logits - m)
    pr_ref[...] = ex / jnp.sum(ex, axis=-1, keepdims=True)


def _final(h, p, b, s, w):
    n = h.shape[0]
    d = w.shape[1]
    return pl.pallas_call(
        _final_body,
        out_shape=(jax.ShapeDtypeStruct((n, d), jnp.float32),
                   jax.ShapeDtypeStruct((n, d), jnp.float32)),
    )(h, p, b, s, w)


@jax.jit
def kernel(x, edge_index, W1, b1, eps1, W2, b2, eps2, W3, b3, eps3):
    pad = jnp.concatenate(
        [jnp.zeros((1, _EP - _E), jnp.int32),
         jnp.full((1, _EP - _E), _NP - 1, jnp.int32)], axis=0)
    ei = jnp.concatenate([edge_index, pad], axis=1).reshape(
        2, _EP // _CH, _CH)
    s1 = jnp.reshape(1.0 + eps1, (1, 1))
    s2 = jnp.reshape(1.0 + eps2, (1, 1))
    s3 = jnp.reshape(1.0 + eps3, (1, 1))

    agg128 = _make_agg(128)

    y1 = _matmul(x, W1)
    p1 = agg128(y1, ei)
    y2 = _combine_mm(y1, p1, b1.reshape(1, -1), s1, W2)
    p2 = agg128(y2, ei)
    h2 = _combine(y2, p2, b2.reshape(1, -1), s2)
    p3 = agg128(h2, ei)
    logits, probs = _final(h2, p3, b3.reshape(1, -1), s3, W3)
    return (logits, probs)
